# baseline (speedup 1.0000x reference)
import functools

import jax
import jax.numpy as jnp
from jax import lax
from jax.experimental import pallas as pl
from jax.experimental.pallas import tpu as pltpu

D = 100000
V = 2000
K = 64
B = 128
_beta = 0.05
_mu = 0.05
_eta = 0.1
_eps = 1e-06
_rho = 1.0 / (1 + 5) ** 0.9
_F32 = jnp.float32
_PREC = lax.Precision.HIGHEST


def _body(idx_sref, exp_m_any, bow_ref, en_ref, es_ref, sd_ref, pi_ref,
          out_any, gath, rows_ref, copy_sem, row_sem):
    bulk = pltpu.make_async_copy(exp_m_any, out_any, copy_sem)
    bulk.start()

    def _gstart(j, _):
        pltpu.make_async_copy(
            exp_m_any.at[pl.ds(idx_sref[j], 1)],
            gath.at[pl.ds(j, 1)], row_sem).start()
        return 0

    def _gwait(j, _):
        pltpu.make_async_copy(
            exp_m_any.at[pl.ds(idx_sref[j], 1)],
            gath.at[pl.ds(j, 1)], row_sem).wait()
        return 0

    lax.fori_loop(0, B, _gstart, 0)
    lax.fori_loop(0, B, _gwait, 0)

    bow = bow_ref[...].astype(_F32)
    en = en_ref[...]
    es = es_ref[...]
    sd = sd_ref[...]
    pi = pi_ref[...]
    en_sum = jnp.sum(en, axis=0, keepdims=True)
    es_sum = jnp.sum(es, axis=0, keepdims=True)
    s_cnt = jnp.sum(sd, axis=0, keepdims=True)
    rate_s = (_mu + es) / (_mu * s_cnt + es_sum)
    rate_n = (_beta + en) / (_beta * V + en_sum)
    is_seed = (jnp.sum(sd, axis=1, keepdims=True) > 0).astype(_F32)
    r1 = sd * (pi * rate_s + (1.0 - pi) * rate_n)
    cm = (1.0 - sd) * rate_n
    p = sd * (pi * pi * rate_s + (1.0 - pi) * (1.0 - pi) * rate_n)
    q = (1.0 - is_seed * pi) * cm
    emb = gath[...]
    m_eta = emb + _eta
    s1 = lax.dot_general(m_eta, r1, (((1,), (1,)), ((), ())),
                         precision=_PREC, preferred_element_type=_F32)
    s2 = lax.dot_general(m_eta, cm, (((1,), (1,)), ((), ())),
                         precision=_PREC, preferred_element_type=_F32)
    u1 = bow / (s1 + _eps)
    u2 = bow / (s2 + _eps)
    t = (lax.dot_general(u1, p, (((1,), (0,)), ((), ())),
                         precision=_PREC, preferred_element_type=_F32)
         + lax.dot_general(u2, q, (((1,), (0,)), ((), ())),
                           precision=_PREC, preferred_element_type=_F32))
    rows_ref[...] = (1.0 - _rho) * emb + _rho * (m_eta * t)

    bulk.wait()

    def _sstart(j, _):
        pltpu.make_async_copy(
            rows_ref.at[pl.ds(j, 1)],
            out_any.at[pl.ds(idx_sref[j], 1)], row_sem).start()
        return 0

    def _swait(j, _):
        pltpu.make_async_copy(
            rows_ref.at[pl.ds(j, 1)],
            out_any.at[pl.ds(idx_sref[j], 1)], row_sem).wait()
        return 0

    lax.fori_loop(0, B, _sstart, 0)
    lax.fori_loop(0, B, _swait, 0)


@functools.partial(jax.jit, static_argnames=("interpret",))
def kernel(batch_BOW, batch_indices, exp_m, exp_n, exp_s, seeds_topic_matrix,
           pi, interpret=False):
    grid_spec = pltpu.PrefetchScalarGridSpec(
        num_scalar_prefetch=1,
        grid=(1,),
        in_specs=[
            pl.BlockSpec(memory_space=pl.ANY),
            pl.BlockSpec((B, V), lambda i, idx: (0, 0)),
            pl.BlockSpec((V, K), lambda i, idx: (0, 0)),
            pl.BlockSpec((V, K), lambda i, idx: (0, 0)),
            pl.BlockSpec((V, K), lambda i, idx: (0, 0)),
            pl.BlockSpec((1, K), lambda i, idx: (0, 0)),
        ],
        out_specs=pl.BlockSpec(memory_space=pl.ANY),
        scratch_shapes=[
            pltpu.VMEM((B, K), _F32),
            pltpu.VMEM((B, K), _F32),
            pltpu.SemaphoreType.DMA,
            pltpu.SemaphoreType.DMA,
        ],
    )
    return pl.pallas_call(
        _body,
        grid_spec=grid_spec,
        out_shape=jax.ShapeDtypeStruct((D, K), _F32),
        interpret=interpret,
    )(batch_indices, exp_m, batch_BOW, exp_n, exp_s,
      seeds_topic_matrix, pi.reshape(1, K))


# speedup vs baseline: 17.7653x; 17.7653x over previous
import functools

import jax
import jax.numpy as jnp
from jax import lax
from jax.experimental import pallas as pl
from jax.experimental.pallas import tpu as pltpu

D = 100000
V = 2000
K = 64
B = 128
_beta = 0.05
_mu = 0.05
_eta = 0.1
_eps = 1e-06
_rho = 1.0 / (1 + 5) ** 0.9
_F32 = jnp.float32
_PREC = lax.Precision.HIGHEST


def _body(idx_sref, exp_m_any, bow_ref, en_ref, es_ref, sd_ref, pi_ref,
          out_any, gath, rows_ref, row_sem):
    def _gstart(j, _):
        pltpu.make_async_copy(
            exp_m_any.at[pl.ds(idx_sref[j], 1)],
            gath.at[pl.ds(j, 1)], row_sem).start()
        return 0

    def _gwait(j, _):
        pltpu.make_async_copy(
            exp_m_any.at[pl.ds(idx_sref[j], 1)],
            gath.at[pl.ds(j, 1)], row_sem).wait()
        return 0

    lax.fori_loop(0, B, _gstart, 0)
    lax.fori_loop(0, B, _gwait, 0)

    bow = bow_ref[...].astype(_F32)
    en = en_ref[...]
    es = es_ref[...]
    sd = sd_ref[...]
    pi = pi_ref[...]
    en_sum = jnp.sum(en, axis=0, keepdims=True)
    es_sum = jnp.sum(es, axis=0, keepdims=True)
    s_cnt = jnp.sum(sd, axis=0, keepdims=True)
    rate_s = (_mu + es) / (_mu * s_cnt + es_sum)
    rate_n = (_beta + en) / (_beta * V + en_sum)
    is_seed = (jnp.sum(sd, axis=1, keepdims=True) > 0).astype(_F32)
    r1 = sd * (pi * rate_s + (1.0 - pi) * rate_n)
    cm = (1.0 - sd) * rate_n
    p = sd * (pi * pi * rate_s + (1.0 - pi) * (1.0 - pi) * rate_n)
    q = (1.0 - is_seed * pi) * cm
    emb = gath[...]
    m_eta = emb + _eta
    s1 = lax.dot_general(m_eta, r1, (((1,), (1,)), ((), ())),
                         precision=_PREC, preferred_element_type=_F32)
    s2 = lax.dot_general(m_eta, cm, (((1,), (1,)), ((), ())),
                         precision=_PREC, preferred_element_type=_F32)
    u1 = bow / (s1 + _eps)
    u2 = bow / (s2 + _eps)
    t = (lax.dot_general(u1, p, (((1,), (0,)), ((), ())),
                         precision=_PREC, preferred_element_type=_F32)
         + lax.dot_general(u2, q, (((1,), (0,)), ((), ())),
                           precision=_PREC, preferred_element_type=_F32))
    rows_ref[...] = (1.0 - _rho) * emb + _rho * (m_eta * t)

    def _sstart(j, _):
        pltpu.make_async_copy(
            rows_ref.at[pl.ds(j, 1)],
            out_any.at[pl.ds(idx_sref[j], 1)], row_sem).start()
        return 0

    def _swait(j, _):
        pltpu.make_async_copy(
            rows_ref.at[pl.ds(j, 1)],
            out_any.at[pl.ds(idx_sref[j], 1)], row_sem).wait()
        return 0

    lax.fori_loop(0, B, _sstart, 0)
    lax.fori_loop(0, B, _swait, 0)


@functools.partial(jax.jit, static_argnames=("interpret",))
def kernel(batch_BOW, batch_indices, exp_m, exp_n, exp_s, seeds_topic_matrix,
           pi, interpret=False):
    grid_spec = pltpu.PrefetchScalarGridSpec(
        num_scalar_prefetch=1,
        grid=(1,),
        in_specs=[
            pl.BlockSpec(memory_space=pl.ANY),
            pl.BlockSpec((B, V), lambda i, idx: (0, 0)),
            pl.BlockSpec((V, K), lambda i, idx: (0, 0)),
            pl.BlockSpec((V, K), lambda i, idx: (0, 0)),
            pl.BlockSpec((V, K), lambda i, idx: (0, 0)),
            pl.BlockSpec((1, K), lambda i, idx: (0, 0)),
        ],
        out_specs=pl.BlockSpec(memory_space=pl.ANY),
        scratch_shapes=[
            pltpu.VMEM((B, K), _F32),
            pltpu.VMEM((B, K), _F32),
            pltpu.SemaphoreType.DMA,
        ],
    )
    return pl.pallas_call(
        _body,
        grid_spec=grid_spec,
        out_shape=jax.ShapeDtypeStruct((D, K), _F32),
        input_output_aliases={1: 0},
        interpret=interpret,
    )(batch_indices, exp_m, batch_BOW, exp_n, exp_s,
      seeds_topic_matrix, pi.reshape(1, K))


# speedup vs baseline: 17.9543x; 1.0106x over previous
import functools

import jax
import jax.numpy as jnp
from jax import lax
from jax.experimental import pallas as pl
from jax.experimental.pallas import tpu as pltpu

D = 100000
V = 2000
K = 64
B = 128
_beta = 0.05
_mu = 0.05
_eta = 0.1
_eps = 1e-06
_rho = 1.0 / (1 + 5) ** 0.9
_F32 = jnp.float32
_PREC = lax.Precision.HIGHEST


def _body(idx_sref, exp_m_any, bow_ref, en_ref, es_ref, sd_ref, pi_ref,
          out_any, gath, rows_ref, row_sem):
    for j in range(B):
        pltpu.make_async_copy(
            exp_m_any.at[pl.ds(idx_sref[j], 1)],
            gath.at[pl.ds(j, 1)], row_sem).start()
    for j in range(B):
        pltpu.make_async_copy(
            exp_m_any.at[pl.ds(idx_sref[j], 1)],
            gath.at[pl.ds(j, 1)], row_sem).wait()

    bow = bow_ref[...].astype(_F32)
    en = en_ref[...]
    es = es_ref[...]
    sd = sd_ref[...]
    pi = pi_ref[...]
    en_sum = jnp.sum(en, axis=0, keepdims=True)
    es_sum = jnp.sum(es, axis=0, keepdims=True)
    s_cnt = jnp.sum(sd, axis=0, keepdims=True)
    rate_s = (_mu + es) / (_mu * s_cnt + es_sum)
    rate_n = (_beta + en) / (_beta * V + en_sum)
    is_seed = (jnp.sum(sd, axis=1, keepdims=True) > 0).astype(_F32)
    r1 = sd * (pi * rate_s + (1.0 - pi) * rate_n)
    cm = (1.0 - sd) * rate_n
    p = sd * (pi * pi * rate_s + (1.0 - pi) * (1.0 - pi) * rate_n)
    q = (1.0 - is_seed * pi) * cm
    emb = gath[...]
    m_eta = emb + _eta
    s1 = lax.dot_general(m_eta, r1, (((1,), (1,)), ((), ())),
                         precision=_PREC, preferred_element_type=_F32)
    s2 = lax.dot_general(m_eta, cm, (((1,), (1,)), ((), ())),
                         precision=_PREC, preferred_element_type=_F32)
    u1 = bow / (s1 + _eps)
    u2 = bow / (s2 + _eps)
    t = (lax.dot_general(u1, p, (((1,), (0,)), ((), ())),
                         precision=_PREC, preferred_element_type=_F32)
         + lax.dot_general(u2, q, (((1,), (0,)), ((), ())),
                           precision=_PREC, preferred_element_type=_F32))
    rows_ref[...] = (1.0 - _rho) * emb + _rho * (m_eta * t)

    for j in range(B):
        pltpu.make_async_copy(
            rows_ref.at[pl.ds(j, 1)],
            out_any.at[pl.ds(idx_sref[j], 1)], row_sem).start()
    for j in range(B):
        pltpu.make_async_copy(
            rows_ref.at[pl.ds(j, 1)],
            out_any.at[pl.ds(idx_sref[j], 1)], row_sem).wait()


@functools.partial(jax.jit, static_argnames=("interpret",))
def kernel(batch_BOW, batch_indices, exp_m, exp_n, exp_s, seeds_topic_matrix,
           pi, interpret=False):
    grid_spec = pltpu.PrefetchScalarGridSpec(
        num_scalar_prefetch=1,
        grid=(1,),
        in_specs=[
            pl.BlockSpec(memory_space=pl.ANY),
            pl.BlockSpec((B, V), lambda i, idx: (0, 0)),
            pl.BlockSpec((V, K), lambda i, idx: (0, 0)),
            pl.BlockSpec((V, K), lambda i, idx: (0, 0)),
            pl.BlockSpec((V, K), lambda i, idx: (0, 0)),
            pl.BlockSpec((1, K), lambda i, idx: (0, 0)),
        ],
        out_specs=pl.BlockSpec(memory_space=pl.ANY),
        scratch_shapes=[
            pltpu.VMEM((B, K), _F32),
            pltpu.VMEM((B, K), _F32),
            pltpu.SemaphoreType.DMA,
        ],
    )
    return pl.pallas_call(
        _body,
        grid_spec=grid_spec,
        out_shape=jax.ShapeDtypeStruct((D, K), _F32),
        input_output_aliases={1: 0},
        interpret=interpret,
    )(batch_indices, exp_m, batch_BOW, exp_n, exp_s,
      seeds_topic_matrix, pi.reshape(1, K))
